# baseline (speedup 1.0000x reference)
import functools

import jax
import jax.numpy as jnp
from jax import lax
from jax.experimental import pallas as pl
from jax.experimental.pallas import tpu as pltpu
from jax.experimental.pallas import tpu_sc as plsc

N = 10000
E = 320000
D_IN = 128
D_HID = 256
D_OUT = 128
D_ATT = 64

NC = 2
NS = 16
NP = 2
TH = N // NP
XR = 5120
EPS = E // NS
CH = 32
NBG = 64
G = 10
NB = NBG * G
EPP = NB * CH
NBD = EPS // CH
AC = 640
ACL = N - AC * (NS - 1)
XC = XR // NS
ZR = 16

_BS = 1000


def _agg_body(ones_mode, *refs):
    if ones_mode:
        (ones_h, tgtw, zb_h, xpart,
         tgt_vm, buf0, zbuf, xagg_sp, gsem0) = refs
    else:
        (h_tbl, srcw, tgtw, zb_h, xpart,
         src_vm, tgt_vm, buf0, buf1, buf2, buf3, zbuf, xagg_sp,
         gsem0, gsem1, gsem2, gsem3, ssem0, ssem1, ssem2, ssem3) = refs
        bufs = (buf0, buf1, buf2, buf3)
        gsems = (gsem0, gsem1, gsem2, gsem3)
        ssems = (ssem0, ssem1, ssem2, ssem3)
    cid = lax.axis_index("c")
    sid = lax.axis_index("s")
    base = sid * XC

    pltpu.sync_copy(zb_h, zbuf)
    if ones_mode:
        pltpu.sync_copy(ones_h, buf0)

    for t in range(NP):
        for z in range(XC // ZR):
            pltpu.sync_copy(zbuf, xagg_sp.at[pl.ds(base + z * ZR, ZR)])
        plsc.subcore_barrier()

        if not ones_mode:
            H = CH // 2

            def gather_start(j, buf, sem):
                pltpu.async_copy(h_tbl.at[src_vm.at[j, pl.ds(0, H)]],
                                 buf.at[pl.ds(0, H)], sem)
                pltpu.async_copy(h_tbl.at[src_vm.at[j, pl.ds(H, H)]],
                                 buf.at[pl.ds(H, H)], sem)

            def gather_wait(j, buf, sem):
                pltpu.make_async_copy(h_tbl.at[src_vm.at[j, pl.ds(0, H)]],
                                      buf.at[pl.ds(0, H)], sem).wait()
                pltpu.make_async_copy(h_tbl.at[src_vm.at[j, pl.ds(H, H)]],
                                      buf.at[pl.ds(H, H)], sem).wait()

        def scatter_start(j, buf, sem):
            pltpu.async_copy(buf, xagg_sp.at[tgt_vm.at[j]], sem, add=True)

        def scatter_wait(j, buf, sem):
            pltpu.make_async_copy(buf, xagg_sp.at[tgt_vm.at[j]], sem).wait()

        for g in range(G):
            if ones_mode:
                pltpu.sync_copy(tgtw.at[t, sid, pl.ds(g * NBG, NBG)], tgt_vm)

                def dstep(j, carry):
                    scatter_start(j, buf0, gsem0)
                    return carry

                lax.fori_loop(0, NBG, dstep, 0)

                def ddrain(j, carry):
                    scatter_wait(j, buf0, gsem0)
                    return carry

                lax.fori_loop(0, NBG, ddrain, 0)
                continue

            pltpu.sync_copy(srcw.at[cid, sid, pl.ds(g * NBG, NBG)], src_vm)
            pltpu.sync_copy(tgtw.at[t, sid, pl.ds(g * NBG, NBG)], tgt_vm)

            gather_start(0, buf0, gsem0)
            gather_start(1, buf1, gsem1)
            gather_start(2, buf2, gsem2)

            def step(k, carry):
                for p in range(4):
                    j = 4 * k + p
                    q = (p + 3) % 4

                    @pl.when(j - 1 >= 0)
                    def _():
                        scatter_wait(j - 1, bufs[q], ssems[q])

                    @pl.when(j + 3 < NBG)
                    def _():
                        gather_start(j + 3, bufs[q], gsems[q])

                    gather_wait(j, bufs[p], gsems[p])
                    scatter_start(j, bufs[p], ssems[p])
                return carry

            lax.fori_loop(0, NBG // 4, step, 0)
            scatter_wait(NBG - 1, bufs[(NBG - 1) % 4], ssems[(NBG - 1) % 4])

        plsc.subcore_barrier()

        @pl.when(sid < NS - 1)
        def _():
            pltpu.sync_copy(xagg_sp.at[pl.ds(base, XC)],
                            xpart.at[cid, t, pl.ds(base, XC)])

        @pl.when(sid == NS - 1)
        def _():
            pltpu.sync_copy(xagg_sp.at[pl.ds(base, TH - (NS - 1) * XC)],
                            xpart.at[cid, t, pl.ds(base, TH - (NS - 1) * XC)])


_sc_mesh = plsc.VectorSubcoreMesh(core_axis_name="c", subcore_axis_name="s",
                                  num_cores=NC, num_subcores=NS)

_agg = pl.kernel(
    functools.partial(_agg_body, False),
    out_type=jax.ShapeDtypeStruct((NC, NP, TH, 128), jnp.float32),
    mesh=_sc_mesh,
    scratch_types=[
        pltpu.VMEM((NBG, CH), jnp.int32),
        pltpu.VMEM((NBG, CH), jnp.int32),
        pltpu.VMEM((CH, 128), jnp.float32),
        pltpu.VMEM((CH, 128), jnp.float32),
        pltpu.VMEM((CH, 128), jnp.float32),
        pltpu.VMEM((CH, 128), jnp.float32),
        pltpu.VMEM((ZR, 128), jnp.float32),
        pltpu.VMEM_SHARED((XR, 128), jnp.float32),
        pltpu.SemaphoreType.DMA,
        pltpu.SemaphoreType.DMA,
        pltpu.SemaphoreType.DMA,
        pltpu.SemaphoreType.DMA,
        pltpu.SemaphoreType.DMA,
        pltpu.SemaphoreType.DMA,
        pltpu.SemaphoreType.DMA,
        pltpu.SemaphoreType.DMA,
    ],
    name="gcn_agg",
)

_aggd = pl.kernel(
    functools.partial(_agg_body, True),
    out_type=jax.ShapeDtypeStruct((NC, NP, TH, 128), jnp.float32),
    mesh=_sc_mesh,
    scratch_types=[
        pltpu.VMEM((NBG, CH), jnp.int32),
        pltpu.VMEM((CH, 128), jnp.float32),
        pltpu.VMEM((ZR, 128), jnp.float32),
        pltpu.VMEM_SHARED((XR, 128), jnp.float32),
        pltpu.SemaphoreType.DMA,
    ],
    name="gcn_deg",
)


def _fc1_body(x_ref, w1t_ref, b1_ref, h_ref, hsp_ref):
    h = jnp.dot(x_ref[...], w1t_ref[...], preferred_element_type=jnp.float32)
    h = jnp.maximum(h + b1_ref[...], 0.0)
    h_ref[...] = h
    for c in range(NC):
        hsp_ref[c] = h[:, 128 * c:128 * (c + 1)]


_fc1 = pl.pallas_call(
    _fc1_body,
    grid=(N // _BS,),
    in_specs=[
        pl.BlockSpec((_BS, D_IN), lambda i: (i, 0)),
        pl.BlockSpec((D_IN, D_HID), lambda i: (0, 0)),
        pl.BlockSpec((1, D_HID), lambda i: (0, 0)),
    ],
    out_specs=[
        pl.BlockSpec((_BS, D_HID), lambda i: (i, 0)),
        pl.BlockSpec((NC, _BS, 128), lambda i: (0, i, 0)),
    ],
    out_shape=(
        jax.ShapeDtypeStruct((N, D_HID), jnp.float32),
        jax.ShapeDtypeStruct((NC, N, 128), jnp.float32),
    ),
)


def _blend(h_ref, p_ref, degp_ref):
    dinv = 0.3 / (degp_ref[:, :1] + 1.0)
    xa = jnp.concatenate([p_ref[0, 0], p_ref[1, 0]], axis=1)
    return 0.7 * h_ref[...] + xa * dinv


def _blend_body(h_ref, p_ref, degp_ref, h1_ref, h1sp_ref):
    h1 = _blend(h_ref, p_ref, degp_ref)
    h1_ref[...] = h1
    for c in range(NC):
        h1sp_ref[c] = h1[:, 128 * c:128 * (c + 1)]


_p_spec = pl.BlockSpec((NC, 1, _BS, 128), lambda i: (0, i // 5, i % 5, 0))

_blend1 = pl.pallas_call(
    _blend_body,
    grid=(N // _BS,),
    in_specs=[
        pl.BlockSpec((_BS, D_HID), lambda i: (i, 0)),
        _p_spec,
        pl.BlockSpec((_BS, 128), lambda i: (i, 0)),
    ],
    out_specs=[
        pl.BlockSpec((_BS, D_HID), lambda i: (i, 0)),
        pl.BlockSpec((NC, _BS, 128), lambda i: (0, i, 0)),
    ],
    out_shape=(
        jax.ShapeDtypeStruct((N, D_HID), jnp.float32),
        jax.ShapeDtypeStruct((NC, N, 128), jnp.float32),
    ),
)


def _final_body(h_ref, p_ref, degp_ref, w2t_ref, b2_ref, wa1t_ref, ba1_ref,
                wa2t_ref, ba2_ref, out_ref, attn_ref):
    h2 = _blend(h_ref, p_ref, degp_ref)
    out = jnp.dot(h2, w2t_ref[...], preferred_element_type=jnp.float32)
    out = jnp.maximum(out + b2_ref[...], 0.0)
    out_ref[...] = out
    a = jnp.dot(out, wa1t_ref[...], preferred_element_type=jnp.float32)
    a = jnp.maximum(a + ba1_ref[...], 0.0)
    t = jnp.dot(a, wa2t_ref[...], preferred_element_type=jnp.float32)
    attn_ref[...] = jax.nn.sigmoid(t + ba2_ref[...])


_final = pl.pallas_call(
    _final_body,
    grid=(N // _BS,),
    in_specs=[
        pl.BlockSpec((_BS, D_HID), lambda i: (i, 0)),
        _p_spec,
        pl.BlockSpec((_BS, 128), lambda i: (i, 0)),
        pl.BlockSpec((D_HID, D_OUT), lambda i: (0, 0)),
        pl.BlockSpec((1, D_OUT), lambda i: (0, 0)),
        pl.BlockSpec((D_OUT, D_ATT), lambda i: (0, 0)),
        pl.BlockSpec((1, D_ATT), lambda i: (0, 0)),
        pl.BlockSpec((D_ATT, 1), lambda i: (0, 0)),
        pl.BlockSpec((1, 1), lambda i: (0, 0)),
    ],
    out_specs=[
        pl.BlockSpec((_BS, D_OUT), lambda i: (i, 0)),
        pl.BlockSpec((_BS, 1), lambda i: (i, 0)),
    ],
    out_shape=(
        jax.ShapeDtypeStruct((N, D_OUT), jnp.float32),
        jax.ShapeDtypeStruct((N, 1), jnp.float32),
    ),
)


def kernel(x, edge_index, W1, b1, W2, b2, Wa1, ba1, Wa2, ba2):
    pad = jnp.zeros((NS, EPP - EPS), jnp.int32)
    src = jnp.concatenate(
        [edge_index[0].astype(jnp.int32).reshape(NS, EPS), pad],
        axis=1).reshape(NS, NB, CH)
    srcw = (src[None] +
            (N * jnp.arange(NC, dtype=jnp.int32)).reshape(NC, 1, 1, 1))
    tgt = edge_index[1].astype(jnp.int32)
    tgtp = jnp.concatenate(
        [tgt.reshape(NS, EPS), pad + N], axis=1).reshape(NS, NB, CH)
    dummy = TH + (
        jnp.arange(EPP * NS, dtype=jnp.int32).reshape(NS, NB, CH) % (XR - TH))
    tgtw = jnp.stack([
        jnp.where((tgtp >= TH * t) & (tgtp < TH * (t + 1)), tgtp - TH * t, dummy)
        for t in range(NP)
    ])

    h, hsp = _fc1(x, W1.T, b1.reshape(1, -1))

    zb = jnp.zeros((ZR, 128), jnp.float32)

    ones_b = jnp.ones((CH, 128), jnp.float32)
    degx = _aggd(ones_b, tgtw, zb)
    degn = degx[0].reshape(N, 128)

    xp1 = _agg(hsp.reshape(NC * N, 128), srcw, tgtw, zb)
    h1, h1sp = _blend1(h, xp1, degn)

    xp2 = _agg(h1sp.reshape(NC * N, 128), srcw, tgtw, zb)

    out, attn = _final(h1, xp2, degn,
                       W2.T, b2.reshape(1, -1), Wa1.T, ba1.reshape(1, -1),
                       Wa2.T, ba2.reshape(1, -1))
    return out, attn


# speedup vs baseline: 2.1237x; 2.1237x over previous
import functools

import jax
import jax.numpy as jnp
from jax import lax
from jax.experimental import pallas as pl
from jax.experimental.pallas import tpu as pltpu
from jax.experimental.pallas import tpu_sc as plsc

N = 10000
E = 320000
D_IN = 128
D_HID = 256
D_OUT = 128
D_ATT = 64

NC = 2
NS = 16
NP = 2
TH = N // NP
XR = 5120
EPS = E // NS
CH = 32
NBG = 64
G = 10
NB = NBG * G
EPP = NB * CH
NBD = EPS // CH
AC = 640
ACL = N - AC * (NS - 1)
XC = XR // NS
ZR = 16

_BS = 1000


def _agg_body(ones_mode, *refs):
    if ones_mode:
        (ones_h, tgtw, zb_h, xpart,
         tgt_vm, buf0, zbuf, xagg_sp, gsem0) = refs
    else:
        (h_tbl, srcw, tgtw, zb_h, xpart,
         src_vm, tgt_vm, buf0, buf1, buf2, buf3, zbuf, xagg_sp,
         gsem0, gsem1, gsem2, gsem3, ssem0, ssem1, ssem2, ssem3) = refs
        bufs = (buf0, buf1, buf2, buf3)
        gsems = (gsem0, gsem1, gsem2, gsem3)
        ssems = (ssem0, ssem1, ssem2, ssem3)
    cid = lax.axis_index("c")
    sid = lax.axis_index("s")
    base = sid * XC

    pltpu.sync_copy(zb_h, zbuf)
    if ones_mode:
        pltpu.sync_copy(ones_h, buf0)

    for t in range(NP):
        for z in range(XC // ZR):
            pltpu.sync_copy(zbuf, xagg_sp.at[pl.ds(base + z * ZR, ZR)])
        plsc.subcore_barrier()

        def fidx(ref):
            return plsc.Indices(ref, ignored_value=-1)

        if not ones_mode:
            def gather_start(j, buf, sem):
                pltpu.async_copy(h_tbl.at[fidx(src_vm.at[j])], buf, sem)

            def gather_wait(j, buf, sem):
                pltpu.make_async_copy(h_tbl.at[fidx(src_vm.at[j])], buf,
                                      sem).wait()

        def scatter_start(j, buf, sem):
            pltpu.async_copy(buf, xagg_sp.at[fidx(tgt_vm.at[j])], sem, add=True)

        def scatter_wait(j, buf, sem):
            pltpu.make_async_copy(buf, xagg_sp.at[fidx(tgt_vm.at[j])],
                                  sem).wait()

        for g in range(G):
            if ones_mode:
                pltpu.sync_copy(tgtw.at[t, sid, pl.ds(g * NBG, NBG)], tgt_vm)

                def dstep(j, carry):
                    scatter_start(j, buf0, gsem0)
                    return carry

                lax.fori_loop(0, NBG, dstep, 0)

                def ddrain(j, carry):
                    scatter_wait(j, buf0, gsem0)
                    return carry

                lax.fori_loop(0, NBG, ddrain, 0)
                continue

            pltpu.sync_copy(srcw.at[t, cid, sid, pl.ds(g * NBG, NBG)], src_vm)
            pltpu.sync_copy(tgtw.at[t, sid, pl.ds(g * NBG, NBG)], tgt_vm)

            gather_start(0, buf0, gsem0)
            gather_start(1, buf1, gsem1)
            gather_start(2, buf2, gsem2)

            def step(k, carry):
                for p in range(4):
                    j = 4 * k + p
                    q = (p + 3) % 4

                    @pl.when(j - 1 >= 0)
                    def _():
                        scatter_wait(j - 1, bufs[q], ssems[q])

                    @pl.when(j + 3 < NBG)
                    def _():
                        gather_start(j + 3, bufs[q], gsems[q])

                    gather_wait(j, bufs[p], gsems[p])
                    scatter_start(j, bufs[p], ssems[p])
                return carry

            lax.fori_loop(0, NBG // 4, step, 0)
            scatter_wait(NBG - 1, bufs[(NBG - 1) % 4], ssems[(NBG - 1) % 4])

        plsc.subcore_barrier()

        @pl.when(sid < NS - 1)
        def _():
            pltpu.sync_copy(xagg_sp.at[pl.ds(base, XC)],
                            xpart.at[cid, t, pl.ds(base, XC)])

        @pl.when(sid == NS - 1)
        def _():
            pltpu.sync_copy(xagg_sp.at[pl.ds(base, TH - (NS - 1) * XC)],
                            xpart.at[cid, t, pl.ds(base, TH - (NS - 1) * XC)])


_sc_mesh = plsc.VectorSubcoreMesh(core_axis_name="c", subcore_axis_name="s",
                                  num_cores=NC, num_subcores=NS)

_agg = pl.kernel(
    functools.partial(_agg_body, False),
    out_type=jax.ShapeDtypeStruct((NC, NP, TH, 128), jnp.float32),
    mesh=_sc_mesh,
    scratch_types=[
        pltpu.VMEM((NBG, CH), jnp.int32),
        pltpu.VMEM((NBG, CH), jnp.int32),
        pltpu.VMEM((CH, 128), jnp.float32),
        pltpu.VMEM((CH, 128), jnp.float32),
        pltpu.VMEM((CH, 128), jnp.float32),
        pltpu.VMEM((CH, 128), jnp.float32),
        pltpu.VMEM((ZR, 128), jnp.float32),
        pltpu.VMEM_SHARED((XR, 128), jnp.float32),
        pltpu.SemaphoreType.DMA,
        pltpu.SemaphoreType.DMA,
        pltpu.SemaphoreType.DMA,
        pltpu.SemaphoreType.DMA,
        pltpu.SemaphoreType.DMA,
        pltpu.SemaphoreType.DMA,
        pltpu.SemaphoreType.DMA,
        pltpu.SemaphoreType.DMA,
    ],
    name="gcn_agg",
)

_aggd = pl.kernel(
    functools.partial(_agg_body, True),
    out_type=jax.ShapeDtypeStruct((NC, NP, TH, 128), jnp.float32),
    mesh=_sc_mesh,
    scratch_types=[
        pltpu.VMEM((NBG, CH), jnp.int32),
        pltpu.VMEM((CH, 128), jnp.float32),
        pltpu.VMEM((ZR, 128), jnp.float32),
        pltpu.VMEM_SHARED((XR, 128), jnp.float32),
        pltpu.SemaphoreType.DMA,
    ],
    name="gcn_deg",
)


def _fc1_body(x_ref, w1t_ref, b1_ref, h_ref, hsp_ref):
    h = jnp.dot(x_ref[...], w1t_ref[...], preferred_element_type=jnp.float32)
    h = jnp.maximum(h + b1_ref[...], 0.0)
    h_ref[...] = h
    for c in range(NC):
        hsp_ref[c] = h[:, 128 * c:128 * (c + 1)]


_fc1 = pl.pallas_call(
    _fc1_body,
    grid=(N // _BS,),
    in_specs=[
        pl.BlockSpec((_BS, D_IN), lambda i: (i, 0)),
        pl.BlockSpec((D_IN, D_HID), lambda i: (0, 0)),
        pl.BlockSpec((1, D_HID), lambda i: (0, 0)),
    ],
    out_specs=[
        pl.BlockSpec((_BS, D_HID), lambda i: (i, 0)),
        pl.BlockSpec((NC, _BS, 128), lambda i: (0, i, 0)),
    ],
    out_shape=(
        jax.ShapeDtypeStruct((N, D_HID), jnp.float32),
        jax.ShapeDtypeStruct((NC, N, 128), jnp.float32),
    ),
)


def _blend(h_ref, p_ref, degp_ref):
    dinv = 0.3 / (degp_ref[:, :1] + 1.0)
    xa = jnp.concatenate([p_ref[0, 0], p_ref[1, 0]], axis=1)
    return 0.7 * h_ref[...] + xa * dinv


def _blend_body(h_ref, p_ref, degp_ref, h1_ref, h1sp_ref):
    h1 = _blend(h_ref, p_ref, degp_ref)
    h1_ref[...] = h1
    for c in range(NC):
        h1sp_ref[c] = h1[:, 128 * c:128 * (c + 1)]


_p_spec = pl.BlockSpec((NC, 1, _BS, 128), lambda i: (0, i // 5, i % 5, 0))

_blend1 = pl.pallas_call(
    _blend_body,
    grid=(N // _BS,),
    in_specs=[
        pl.BlockSpec((_BS, D_HID), lambda i: (i, 0)),
        _p_spec,
        pl.BlockSpec((_BS, 128), lambda i: (i, 0)),
    ],
    out_specs=[
        pl.BlockSpec((_BS, D_HID), lambda i: (i, 0)),
        pl.BlockSpec((NC, _BS, 128), lambda i: (0, i, 0)),
    ],
    out_shape=(
        jax.ShapeDtypeStruct((N, D_HID), jnp.float32),
        jax.ShapeDtypeStruct((NC, N, 128), jnp.float32),
    ),
)


def _final_body(h_ref, p_ref, degp_ref, w2t_ref, b2_ref, wa1t_ref, ba1_ref,
                wa2t_ref, ba2_ref, out_ref, attn_ref):
    h2 = _blend(h_ref, p_ref, degp_ref)
    out = jnp.dot(h2, w2t_ref[...], preferred_element_type=jnp.float32)
    out = jnp.maximum(out + b2_ref[...], 0.0)
    out_ref[...] = out
    a = jnp.dot(out, wa1t_ref[...], preferred_element_type=jnp.float32)
    a = jnp.maximum(a + ba1_ref[...], 0.0)
    t = jnp.dot(a, wa2t_ref[...], preferred_element_type=jnp.float32)
    attn_ref[...] = jax.nn.sigmoid(t + ba2_ref[...])


_final = pl.pallas_call(
    _final_body,
    grid=(N // _BS,),
    in_specs=[
        pl.BlockSpec((_BS, D_HID), lambda i: (i, 0)),
        _p_spec,
        pl.BlockSpec((_BS, 128), lambda i: (i, 0)),
        pl.BlockSpec((D_HID, D_OUT), lambda i: (0, 0)),
        pl.BlockSpec((1, D_OUT), lambda i: (0, 0)),
        pl.BlockSpec((D_OUT, D_ATT), lambda i: (0, 0)),
        pl.BlockSpec((1, D_ATT), lambda i: (0, 0)),
        pl.BlockSpec((D_ATT, 1), lambda i: (0, 0)),
        pl.BlockSpec((1, 1), lambda i: (0, 0)),
    ],
    out_specs=[
        pl.BlockSpec((_BS, D_OUT), lambda i: (i, 0)),
        pl.BlockSpec((_BS, 1), lambda i: (i, 0)),
    ],
    out_shape=(
        jax.ShapeDtypeStruct((N, D_OUT), jnp.float32),
        jax.ShapeDtypeStruct((N, 1), jnp.float32),
    ),
)


def kernel(x, edge_index, W1, b1, W2, b2, Wa1, ba1, Wa2, ba2):
    pad = jnp.zeros((NS, EPP - EPS), jnp.int32)
    src = jnp.concatenate(
        [edge_index[0].astype(jnp.int32).reshape(NS, EPS), pad],
        axis=1).reshape(NS, NB, CH)
    tgt = edge_index[1].astype(jnp.int32)
    tgtp = jnp.concatenate(
        [tgt.reshape(NS, EPS), pad + N], axis=1).reshape(NS, NB, CH)
    inph = jnp.stack([(tgtp >= TH * t) & (tgtp < TH * (t + 1))
                      for t in range(NP)])
    offs = (N * jnp.arange(NC, dtype=jnp.int32)).reshape(NC, 1, 1, 1)
    srcw = jnp.where(inph[:, None], src[None, None] + offs, -1)
    tgtw = jnp.stack([jnp.where(inph[t], tgtp - TH * t, -1)
                      for t in range(NP)])

    h, hsp = _fc1(x, W1.T, b1.reshape(1, -1))

    zb = jnp.zeros((ZR, 128), jnp.float32)

    ones_b = jnp.ones((CH, 128), jnp.float32)
    degx = _aggd(ones_b, tgtw, zb)
    degn = degx[0].reshape(N, 128)

    xp1 = _agg(hsp.reshape(NC * N, 128), srcw, tgtw, zb)
    h1, h1sp = _blend1(h, xp1, degn)

    xp2 = _agg(h1sp.reshape(NC * N, 128), srcw, tgtw, zb)

    out, attn = _final(h1, xp2, degn,
                       W2.T, b2.reshape(1, -1), Wa1.T, ba1.reshape(1, -1),
                       Wa2.T, ba2.reshape(1, -1))
    return out, attn
